# baseline (speedup 1.0000x reference)
import functools

import jax
import jax.numpy as jnp
from jax import lax
from jax.experimental import pallas as pl
from jax.experimental.pallas import tpu as pltpu
from jax.experimental.pallas import tpu_sc as plsc

N = 10000
N_EXT = 10112
E = 320000
NC, NS = 2, 16
NW = NC * NS
CH = 128
NCHUNK = 80
EPT = NCHUNK * CH
E_PAD = NW * EPT
ROWS2D = E_PAD // CH
RPT = N_EXT // NS

_MESH = plsc.VectorSubcoreMesh(core_axis_name="c", subcore_axis_name="s")


@functools.partial(
    pl.kernel,
    out_type=jax.ShapeDtypeStruct((NW, N_EXT), jnp.float32),
    mesh=_MESH,
    scratch_types=[
        pltpu.VMEM((N_EXT,), jnp.float32),
        pltpu.VMEM((NCHUNK, CH), jnp.int32),
    ],
    compiler_params=pltpu.CompilerParams(
        needs_layout_passes=False, use_tc_tiling_on_sc=False),
)
def _deg(dst_hbm, degp_hbm, deg_v, didx_v):
    cid = lax.axis_index("c")
    sid = lax.axis_index("s")
    wid = sid * NC + cid

    zeros = jnp.zeros((16,), jnp.float32)

    def zbody(i, c):
        deg_v[pl.ds(i * 16, 16)] = zeros
        return c

    lax.fori_loop(0, N_EXT // 16, zbody, 0)

    pltpu.sync_copy(dst_hbm.at[pl.ds(wid * NCHUNK, NCHUNK)], didx_v)

    ones = jnp.full((16,), 1.0, jnp.float32)

    def ebody(i, c):
        idx = didx_v[i // (CH // 16), pl.ds((i % (CH // 16)) * 16, 16)]
        plsc.addupdate_scatter(deg_v, [idx], ones)
        return c

    lax.fori_loop(0, EPT // 16, ebody, 0)

    pltpu.sync_copy(deg_v, degp_hbm.at[wid])


NBUF = 10
GRP = NCHUNK // NBUF


def _make_agg(F):
    @functools.partial(
        pl.kernel,
        out_type=jax.ShapeDtypeStruct((N_EXT, 2 * F), jnp.float32),
        mesh=_MESH,
        scratch_types=(
            [
                pltpu.VMEM_SHARED((N_EXT, F), jnp.float32),
                pltpu.VMEM((NCHUNK, CH), jnp.int32),
                pltpu.VMEM((NCHUNK, CH), jnp.int32),
            ]
            + [pltpu.VMEM((CH, F), jnp.float32) for _ in range(NBUF)]
            + [pltpu.SemaphoreType.DMA for _ in range(2 * NBUF)]
        ),
        compiler_params=pltpu.CompilerParams(use_tc_tiling_on_sc=False),
    )
    def agg(u_hbm, src_hbm, dst_hbm, p_hbm, acc, sidx, didx, *bufs_sems):
        rows = bufs_sems[:NBUF]
        gs = bufs_sems[NBUF:2 * NBUF]
        ss = bufs_sems[2 * NBUF:]
        cid = lax.axis_index("c")
        sid = lax.axis_index("s")
        wid = sid * NC + cid
        rbase = sid * RPT

        pltpu.sync_copy(u_hbm.at[pl.ds(rbase, RPT)], acc.at[pl.ds(rbase, RPT)])

        pltpu.sync_copy(src_hbm.at[pl.ds(wid * NCHUNK, NCHUNK)], sidx)
        pltpu.sync_copy(dst_hbm.at[pl.ds(wid * NCHUNK, NCHUNK)], didx)

        plsc.subcore_barrier()

        for b in range(NBUF):
            pltpu.async_copy(u_hbm.at[sidx.at[b]], rows[b], gs[b])

        def round_(g, c):
            for b in range(NBUF):
                j = g * NBUF + b
                pltpu.make_async_copy(u_hbm.at[sidx.at[j]], rows[b], gs[b]).wait()
                pltpu.async_copy(rows[b], acc.at[didx.at[j]], ss[b], add=True)
            for b in range(NBUF):
                pltpu.make_async_copy(rows[b], acc.at[didx.at[b]], ss[b]).wait()

                @pl.when(g + 1 < GRP)
                def _():
                    jn = (g + 1) * NBUF + b
                    pltpu.async_copy(u_hbm.at[sidx.at[jn]], rows[b], gs[b])

            return c

        lax.fori_loop(0, GRP, round_, 0)

        plsc.subcore_barrier()

        pltpu.sync_copy(acc.at[pl.ds(rbase, RPT)],
                        p_hbm.at[pl.ds(rbase, RPT), pl.ds(cid * F, F)])

    return agg


_agg8 = _make_agg(8)
_agg16 = _make_agg(16)
_agg40 = _make_agg(40)


_PAD = N_EXT - N


def _tc_first_body(degp_ref, x_ref, w_ref, dinv_ref, u_ref):
    deg = jnp.dot(jnp.ones((1, NW), jnp.float32), degp_ref[...],
                  preferred_element_type=jnp.float32)
    col = lax.broadcasted_iota(jnp.int32, (1, N_EXT), 1)
    dinv = jnp.where(col < N, lax.rsqrt(deg + 1.0), 0.0)
    dinv_ref[...] = dinv
    h = lax.dot_general(w_ref[...], x_ref[...], (((1,), (1,)), ((), ())),
                        preferred_element_type=jnp.float32)
    full = jnp.concatenate(
        [h, jnp.zeros((w_ref.shape[0], _PAD), jnp.float32)], axis=1)
    u_ref[...] = dinv * full


def _tc_mid_body(dinv_ref, p_ref, u_ref, b_ref, w_ref, un_ref):
    dinv = dinv_ref[...]
    fv = u_ref.shape[0]
    p = p_ref[...]
    s = p[:fv] + p[fv:] - u_ref[...]
    pre = dinv * s + b_ref[...]
    h = jnp.maximum(pre, 0.0)
    un_ref[...] = dinv * jnp.dot(w_ref[...], h,
                                 preferred_element_type=jnp.float32)


def _tc_last_body(dinv_ref, p_ref, u_ref, b_ref, o_ref):
    dinv = dinv_ref[...][:, :N]
    fv = u_ref.shape[0]
    p = p_ref[:, pl.ds(0, N)]
    s = p[:fv] + p[fv:] - u_ref[:, pl.ds(0, N)]
    pre = dinv * s + b_ref[...]
    m = jnp.max(pre, axis=0, keepdims=True)
    lse = jnp.log(jnp.sum(jnp.exp(pre - m), axis=0, keepdims=True)) + m
    o_ref[...] = pre - lse


def _tc_first(degp, x, wT):
    return pl.pallas_call(
        _tc_first_body,
        out_shape=(
            jax.ShapeDtypeStruct((1, N_EXT), jnp.float32),
            jax.ShapeDtypeStruct((wT.shape[0], N_EXT), jnp.float32),
        ),
    )(degp, x, wT)


def _tc_mid(dinv, pT, uT, bT, wT):
    return pl.pallas_call(
        _tc_mid_body,
        out_shape=jax.ShapeDtypeStruct((wT.shape[0], N_EXT), jnp.float32),
    )(dinv, pT, uT, bT, wT)


def _tc_last(dinv, pT, uT, bT):
    return pl.pallas_call(
        _tc_last_body,
        out_shape=jax.ShapeDtypeStruct((uT.shape[0], N), jnp.float32),
    )(dinv, pT, uT, bT)


def kernel(x, adj, W1, b1, W2, b2, W3, b3, W4, b4):
    src = adj[0].astype(jnp.int32)
    dst = adj[1].astype(jnp.int32)
    pad = (jnp.arange(E_PAD - E, dtype=jnp.int32) % _PAD) + N
    src2 = jnp.concatenate([src, pad]).reshape(ROWS2D, CH)
    dst2 = jnp.concatenate([dst, pad]).reshape(ROWS2D, CH)

    degp = _deg(dst2)

    dinv, u1T = _tc_first(degp, x, W1.T)

    p1 = _agg8(u1T.T, src2, dst2)
    u2T = _tc_mid(dinv, p1.T, u1T, b1.reshape(-1, 1), W2.T)

    p2 = _agg16(u2T.T, src2, dst2)
    u3T = _tc_mid(dinv, p2.T, u2T, b2.reshape(-1, 1), W3.T)

    p3 = _agg8(u3T.T, src2, dst2)
    u4T = _tc_mid(dinv, p3.T, u3T, b3.reshape(-1, 1), W4.T)

    p4 = _agg40(u4T.T, src2, dst2)
    outT = _tc_last(dinv, p4.T, u4T, b4.reshape(-1, 1))
    return outT.T
